# Initial kernel scaffold; baseline (speedup 1.0000x reference)
import jax
import jax.numpy as jnp
from jax.experimental import pallas as pl


def kernel(node_input, edge_src, edge_dst, edge_attr, edge_scalars, fc0_w1, fc0_w2, fc1_w1, fc1_w2):
    raise NotImplementedError("write your pallas kernel here")


# speedup vs baseline: 1.8479x; 1.8479x over previous
import functools

import jax
import jax.numpy as jnp
import numpy as np
from jax import lax
from jax.experimental import pallas as pl
from jax.experimental.pallas import tpu as pltpu
from jax.experimental.pallas import tpu_sc as plsc

SILU_NORM = 1.679177
NUM_NEIGHBORS = 32.0

NC = 2
NS = 16
NW = NC * NS

LANES = 16


def _silu(x):
    return x / (1.0 + jnp.exp(-x))


def _edge_weights_body(es_ref, attr_ref, w01_ref, w02_ref, w11_ref, w12_ref,
                       o0_ref, o1_ref):
    x = es_ref[...]
    scale = attr_ref[...] * (1.0 / np.sqrt(NUM_NEIGHBORS))

    def fc(w1_ref, w2_ref):
        inv1 = 1.0 / np.sqrt(w1_ref.shape[0])
        inv2 = 1.0 / np.sqrt(w2_ref.shape[0])
        h = lax.dot(x, w1_ref[...] * inv1,
                    precision=lax.Precision.HIGHEST,
                    preferred_element_type=jnp.float32)
        h = SILU_NORM * _silu(h)
        return lax.dot(h, w2_ref[...] * inv2,
                       precision=lax.Precision.HIGHEST,
                       preferred_element_type=jnp.float32)

    o0_ref[...] = fc(w01_ref, w02_ref) * scale
    o1_ref[...] = fc(w11_ref, w12_ref) * scale


def _edge_weights(edge_scalars, edge_attr, fc0_w1, fc0_w2, fc1_w1, fc1_w2):
    E, NB = edge_scalars.shape
    D = fc0_w2.shape[1]
    BE = 2560
    assert E % BE == 0
    grid = (E // BE,)
    full = lambda shape: pl.BlockSpec(shape, lambda i: (0, 0))
    return pl.pallas_call(
        _edge_weights_body,
        grid=grid,
        in_specs=[
            pl.BlockSpec((BE, NB), lambda i: (i, 0)),
            pl.BlockSpec((BE, 1), lambda i: (i, 0)),
            full(fc0_w1.shape), full(fc0_w2.shape),
            full(fc1_w1.shape), full(fc1_w2.shape),
        ],
        out_specs=[
            pl.BlockSpec((BE, D), lambda i: (i, 0)),
            pl.BlockSpec((BE, D), lambda i: (i, 0)),
        ],
        out_shape=[
            jax.ShapeDtypeStruct((E, D), jnp.float32),
            jax.ShapeDtypeStruct((E, D), jnp.float32),
        ],
    )(edge_scalars, edge_attr, fc0_w1, fc0_w2, fc1_w1, fc1_w2)


def _make_sc_conv(NPAD, E, D):
    EPW = E // NW
    CH = 80
    assert EPW % CH == 0
    NCHUNK = EPW // CH
    RPS = NPAD // NS
    SR = 128
    assert RPS % SR == 0
    NSTAGE = RPS // SR

    mesh = plsc.VectorSubcoreMesh(core_axis_name="c", subcore_axis_name="s")

    def body(node_hbm, src_hbm, dst_hbm, w_hbm, out_hbm,
             acc, src_idx, dst_idx, nodes, wts, stage, gsem, wsem):
        cid = lax.axis_index("c")
        sid = lax.axis_index("s")
        wid = sid * NC + cid

        def zrow(i, _):
            for c in range(D // LANES):
                stage[i, pl.ds(c * LANES, LANES)] = jnp.zeros((LANES,),
                                                              jnp.float32)
            return 0
        lax.fori_loop(0, SR, zrow, 0)
        for k in range(NSTAGE):
            pltpu.sync_copy(stage, acc.at[pl.ds(sid * RPS + k * SR, SR)])
        plsc.subcore_barrier()

        base = wid * EPW

        def chunk(i, _):
            off = base + i * CH
            pltpu.sync_copy(src_hbm.at[pl.ds(off, CH)], src_idx)
            pltpu.sync_copy(dst_hbm.at[pl.ds(off, CH)], dst_idx)
            gcp = pltpu.async_copy(node_hbm.at[src_idx], nodes, gsem)
            wcp = pltpu.async_copy(w_hbm.at[pl.ds(off, CH)], wts, wsem)
            gcp.wait()
            wcp.wait()

            def row(r, _):
                for c in range(D // LANES):
                    sl = pl.ds(c * LANES, LANES)
                    nodes[r, sl] = nodes[r, sl] * wts[r, sl]
                return 0
            lax.fori_loop(0, CH, row, 0)
            pltpu.sync_copy(nodes, acc.at[dst_idx], add=True)
            return 0
        lax.fori_loop(0, NCHUNK, chunk, 0)

        plsc.subcore_barrier()
        for k in range(NSTAGE):
            rows = pl.ds(sid * RPS + k * SR, SR)
            pltpu.sync_copy(acc.at[rows], stage)
            pltpu.sync_copy(stage, out_hbm.at[cid, rows])

    return pl.kernel(
        body,
        out_type=jax.ShapeDtypeStruct((NC, NPAD, D), jnp.float32),
        mesh=mesh,
        scratch_types=[
            pltpu.VMEM_SHARED((NPAD, D), jnp.float32),
            pltpu.VMEM((CH,), jnp.int32),
            pltpu.VMEM((CH,), jnp.int32),
            pltpu.VMEM((CH, D), jnp.float32),
            pltpu.VMEM((CH, D), jnp.float32),
            pltpu.VMEM((SR, D), jnp.float32),
            pltpu.SemaphoreType.DMA,
            pltpu.SemaphoreType.DMA,
        ],
    )


def _combine(partials, apply_silu):
    _, N, D = partials.shape
    BR = 2048
    assert N % BR == 0

    def body(p0_ref, p1_ref, o_ref):
        s = p0_ref[0] + p1_ref[0]
        if apply_silu:
            s = SILU_NORM * _silu(s)
        o_ref[...] = s

    return pl.pallas_call(
        body,
        grid=(N // BR,),
        in_specs=[
            pl.BlockSpec((1, BR, D), lambda i: (0, i, 0)),
            pl.BlockSpec((1, BR, D), lambda i: (1, i, 0)),
        ],
        out_specs=pl.BlockSpec((BR, D), lambda i: (i, 0)),
        out_shape=jax.ShapeDtypeStruct((N, D), jnp.float32),
    )(partials, partials)


def kernel(node_input, edge_src, edge_dst, edge_attr, edge_scalars,
           fc0_w1, fc0_w2, fc1_w1, fc1_w2):
    N, D = node_input.shape
    E = edge_src.shape[0]
    NPAD = -(-N // (NS * 128)) * (NS * 128)

    w0, w1 = _edge_weights(edge_scalars, edge_attr,
                           fc0_w1, fc0_w2, fc1_w1, fc1_w2)

    sc_conv = _make_sc_conv(NPAD, E, D)
    src = edge_src.astype(jnp.int32)
    dst = edge_dst.astype(jnp.int32)

    p0 = sc_conv(node_input, src, dst, w0)
    h = _combine(p0, apply_silu=True)
    p1 = sc_conv(h, src, dst, w1)
    return _combine(p1, apply_silu=False)[:N]
